# baseline (speedup 1.0000x reference)
import functools

import jax
import jax.numpy as jnp
import numpy as np
from jax import lax
from jax.experimental import pallas as pl
from jax.experimental.pallas import tpu as pltpu
from jax.experimental.pallas import tpu_sc as plsc

_B = 16384
_K = 50
_D = 128
_L = 16
_NC = 2
_NS = 16
_NW = _NC * _NS
_BPW = _B // _NW
_CHUNK = 128
_NCHUNK = _BPW // _CHUNK
_NDC = _D // _L
_KP = 64
_NBUF = 8
_PAIR = 1
_KUN = 8

_BCAST_DNUMS = lax.GatherDimensionNumbers(
    offset_dims=(), collapsed_slice_dims=(0,), start_index_map=(0,))


def _bcast_lane(vec, lane):
    idx = jnp.full((_L,), lane, jnp.int32).reshape(_L, 1)
    return lax.gather(vec, idx, _BCAST_DNUMS, (1,),
                      mode=lax.GatherScatterMode.PROMISE_IN_BOUNDS)


def _tec_body(idx_hbm, val_hbm, w_hbm, bias_hbm, out_hbm,
              idx_v, val_v, rows_v, out_v, bias_v, sems):
    cidx = lax.axis_index("c")
    sidx = lax.axis_index("s")
    wid = sidx * _NC + cidx
    base = wid * _BPW

    pltpu.sync_copy(bias_hbm, bias_v)
    bias_vecs = tuple(bias_v[pl.ds(c * _L, _L)] for c in range(_NDC))

    def compute_row(i, p, rbuf):
        vrows = [val_v[i, pl.ds(g * _L, _L)] for g in range(_KP // _L)]
        acc = tuple(bias_vecs)
        for g in range(_K // _L + 1):
            nk = _L if g < _K // _L else _K % _L
            kb = p * _K + g * _L
            vrow = vrows[g]

            def kblock(it, acc_t, kb=kb, vrow=vrow):
                k0 = it * _KUN
                a = list(acc_t)
                for kk in range(_KUN):
                    bc = _bcast_lane(vrow, k0 + kk)
                    for c in range(_NDC):
                        a[c] = a[c] + rbuf[kb + k0 + kk,
                                           pl.ds(c * _L, _L)] * bc
                return tuple(a)

            if nk >= _KUN:
                acc = lax.fori_loop(0, nk // _KUN, kblock, acc)
            for k in range(nk - nk % _KUN, nk):
                bc = _bcast_lane(vrow, k)
                acc = tuple(acc[c] + rbuf[kb + k, pl.ds(c * _L, _L)] * bc
                            for c in range(_NDC))
        for c in range(_NDC):
            out_v[i, pl.ds(c * _L, _L)] = acc[c]

    def chunk_body(chunk, _):
        cbase = base + chunk * _CHUNK
        gbase = wid * (_BPW // _PAIR) + chunk * (_CHUNK // _PAIR)
        pltpu.sync_copy(idx_hbm.at[pl.ds(gbase, _CHUNK // _PAIR)], idx_v)
        pltpu.sync_copy(val_hbm.at[pl.ds(cbase, _CHUNK)], val_v)

        for b in range(_NBUF):
            pltpu.async_copy(w_hbm.at[idx_v.at[b]], rows_v.at[b], sems[b])

        def ring_round(r, _):
            g0 = r * _NBUF
            for b in range(_NBUF):
                g = g0 + b
                pltpu.make_async_copy(
                    w_hbm.at[idx_v.at[g]], rows_v.at[b], sems[b]).wait()
                for p in range(_PAIR):
                    compute_row(g * _PAIR + p, p, rows_v.at[b])
                gnext = g + _NBUF

                @pl.when(gnext < _CHUNK // _PAIR)
                def _():
                    pltpu.async_copy(
                        w_hbm.at[idx_v.at[gnext]], rows_v.at[b], sems[b])
            return 0

        lax.fori_loop(0, _CHUNK // _PAIR // _NBUF, ring_round, 0)
        pltpu.sync_copy(out_v, out_hbm.at[pl.ds(cbase, _CHUNK)])
        return 0

    lax.fori_loop(0, _NCHUNK, chunk_body, 0)


@functools.partial(
    pl.kernel,
    out_type=jax.ShapeDtypeStruct((_B, _D), jnp.float32),
    mesh=plsc.VectorSubcoreMesh(core_axis_name="c", subcore_axis_name="s"),
    scratch_types=[
        pltpu.VMEM((_CHUNK // _PAIR, _PAIR * _K), jnp.int32),
        pltpu.VMEM((_CHUNK, _KP), jnp.float32),
        pltpu.VMEM((_NBUF, _PAIR * _K, _D), jnp.float32),
        pltpu.VMEM((_CHUNK, _D), jnp.float32),
        pltpu.VMEM((_D,), jnp.float32),
    ] + [pltpu.SemaphoreType.DMA] * _NBUF,
)
def _ft_sc(idx_hbm, val_hbm, w_hbm, bias_hbm, out_hbm,
           idx_v, val_v, rows_v, out_v, bias_v, *sems):
    _tec_body(idx_hbm, val_hbm, w_hbm, bias_hbm, out_hbm,
              idx_v, val_v, rows_v, out_v, bias_v, sems)


def kernel(feature_indices, feature_values, weight, bias):
    idx_grouped = feature_indices.reshape(_B // _PAIR, _PAIR * _K)
    vals_padded = jnp.pad(feature_values, ((0, 0), (0, _KP - _K)))
    return _ft_sc(idx_grouped, vals_padded, weight, bias)


# speedup vs baseline: 1.1997x; 1.1997x over previous
import functools

import jax
import jax.numpy as jnp
import numpy as np
from jax import lax
from jax.experimental import pallas as pl
from jax.experimental.pallas import tpu as pltpu
from jax.experimental.pallas import tpu_sc as plsc

_B = 16384
_K = 50
_D = 128
_L = 16
_NC = 2
_NS = 16
_NW = _NC * _NS
_BPW = _B // _NW
_CHUNK = 128
_NCHUNK = _BPW // _CHUNK
_NDC = _D // _L
_KP = 64
_NBUF = 4
_PAIR = 1
_KUN = 8

_BCAST_DNUMS = lax.GatherDimensionNumbers(
    offset_dims=(), collapsed_slice_dims=(0,), start_index_map=(0,))


def _bcast_lane(vec, lane):
    idx = jnp.full((_L,), lane, jnp.int32).reshape(_L, 1)
    return lax.gather(vec, idx, _BCAST_DNUMS, (1,),
                      mode=lax.GatherScatterMode.PROMISE_IN_BOUNDS)


def _tec_body(idx_hbm, val_hbm, w_hbm, bias_hbm, out_hbm,
              idx_v, val_v, rows_v, out_v, bias_v, sems, osems):
    cidx = lax.axis_index("c")
    sidx = lax.axis_index("s")
    wid = sidx * _NC + cidx
    base = wid * _BPW

    pltpu.sync_copy(bias_hbm, bias_v)
    bias_vecs = tuple(bias_v[pl.ds(c * _L, _L)] for c in range(_NDC))

    def compute_row(i, p, q, rbuf):
        vrows = [val_v[i, pl.ds(g * _L, _L)] for g in range(_KP // _L)]
        acc = tuple(bias_vecs)
        for g in range(_K // _L + 1):
            nk = _L if g < _K // _L else _K % _L
            kb = p * _K + g * _L
            vrow = vrows[g]

            def kblock(it, acc_t, kb=kb, vrow=vrow):
                k0 = it * _KUN
                a = list(acc_t)
                for kk in range(_KUN):
                    bc = _bcast_lane(vrow, k0 + kk)
                    for c in range(_NDC):
                        a[c] = a[c] + rbuf[kb + k0 + kk,
                                           pl.ds(c * _L, _L)] * bc
                return tuple(a)

            if nk >= _KUN:
                acc = lax.fori_loop(0, nk // _KUN, kblock, acc)
            for k in range(nk - nk % _KUN, nk):
                bc = _bcast_lane(vrow, k)
                acc = tuple(acc[c] + rbuf[kb + k, pl.ds(c * _L, _L)] * bc
                            for c in range(_NDC))
        for c in range(_NDC):
            out_v[q, i, pl.ds(c * _L, _L)] = acc[c]

    for chunk in range(_NCHUNK):
        q = chunk & 1
        cbase = base + chunk * _CHUNK
        gbase = wid * (_BPW // _PAIR) + chunk * (_CHUNK // _PAIR)
        pltpu.sync_copy(idx_hbm.at[pl.ds(gbase, _CHUNK // _PAIR)], idx_v)
        pltpu.sync_copy(val_hbm.at[pl.ds(cbase, _CHUNK)], val_v)

        if chunk >= 2:
            pltpu.make_async_copy(
                out_v.at[q],
                out_hbm.at[pl.ds(base + (chunk - 2) * _CHUNK, _CHUNK)],
                osems[q]).wait()

        for b in range(_NBUF):
            pltpu.async_copy(w_hbm.at[idx_v.at[b]], rows_v.at[b], sems[b])

        def ring_round(r, _, q=q):
            g0 = r * _NBUF
            for b in range(_NBUF):
                g = g0 + b
                pltpu.make_async_copy(
                    w_hbm.at[idx_v.at[g]], rows_v.at[b], sems[b]).wait()
                for p in range(_PAIR):
                    compute_row(g * _PAIR + p, p, q, rows_v.at[b])
                gnext = g + _NBUF

                @pl.when(gnext < _CHUNK // _PAIR)
                def _():
                    pltpu.async_copy(
                        w_hbm.at[idx_v.at[gnext]], rows_v.at[b], sems[b])
            return 0

        lax.fori_loop(0, _CHUNK // _PAIR // _NBUF, ring_round, 0)
        pltpu.async_copy(out_v.at[q], out_hbm.at[pl.ds(cbase, _CHUNK)],
                         osems[q])

    for chunk in (_NCHUNK - 2, _NCHUNK - 1):
        q = chunk & 1
        pltpu.make_async_copy(
            out_v.at[q],
            out_hbm.at[pl.ds(base + chunk * _CHUNK, _CHUNK)],
            osems[q]).wait()


@functools.partial(
    pl.kernel,
    out_type=jax.ShapeDtypeStruct((_B, _D), jnp.float32),
    mesh=plsc.VectorSubcoreMesh(core_axis_name="c", subcore_axis_name="s"),
    scratch_types=[
        pltpu.VMEM((_CHUNK // _PAIR, _PAIR * _K), jnp.int32),
        pltpu.VMEM((_CHUNK, _KP), jnp.float32),
        pltpu.VMEM((_NBUF, _PAIR * _K, _D), jnp.float32),
        pltpu.VMEM((2, _CHUNK, _D), jnp.float32),
        pltpu.VMEM((_D,), jnp.float32),
    ] + [pltpu.SemaphoreType.DMA] * (_NBUF + 2),
)
def _ft_sc(idx_hbm, val_hbm, w_hbm, bias_hbm, out_hbm,
           idx_v, val_v, rows_v, out_v, bias_v, *sems):
    _tec_body(idx_hbm, val_hbm, w_hbm, bias_hbm, out_hbm,
              idx_v, val_v, rows_v, out_v, bias_v,
              sems[:_NBUF], sems[_NBUF:])


def kernel(feature_indices, feature_values, weight, bias):
    idx_grouped = feature_indices.reshape(_B // _PAIR, _PAIR * _K)
    vals_padded = jnp.pad(feature_values, ((0, 0), (0, _KP - _K)))
    return _ft_sc(idx_grouped, vals_padded, weight, bias)


# speedup vs baseline: 1.2170x; 1.0144x over previous
import functools

import jax
import jax.numpy as jnp
import numpy as np
from jax import lax
from jax.experimental import pallas as pl
from jax.experimental.pallas import tpu as pltpu
from jax.experimental.pallas import tpu_sc as plsc

_B = 16384
_K = 50
_D = 128
_L = 16
_NC = 2
_NS = 16
_NW = _NC * _NS
_BPW = _B // _NW
_CHUNK = 128
_NCHUNK = _BPW // _CHUNK
_NDC = _D // _L
_KP = 64
_NBUF = 4
_PAIR = 1
_KUN = 8

_BCAST_DNUMS = lax.GatherDimensionNumbers(
    offset_dims=(), collapsed_slice_dims=(0,), start_index_map=(0,))


def _bcast_lane(vec, lane):
    idx = jnp.full((_L,), lane, jnp.int32).reshape(_L, 1)
    return lax.gather(vec, idx, _BCAST_DNUMS, (1,),
                      mode=lax.GatherScatterMode.PROMISE_IN_BOUNDS)


def _tec_body(idx_hbm, val_hbm, w_hbm, bias_hbm, out_hbm,
              idx_v, val_v, rows_v, out_v, bias_v, sems, osems):
    cidx = lax.axis_index("c")
    sidx = lax.axis_index("s")
    wid = sidx * _NC + cidx
    base = wid * _BPW

    pltpu.sync_copy(bias_hbm, bias_v)
    bias_vecs = tuple(bias_v[pl.ds(c * _L, _L)] for c in range(_NDC))

    def compute_row(i, p, qd, rbuf):
        vrows = [val_v[i, pl.ds(g * _L, _L)] for g in range(_KP // _L)]
        acc = tuple(bias_vecs)
        for g in range(_K // _L + 1):
            nk = _L if g < _K // _L else _K % _L
            kb = p * _K + g * _L
            vrow = vrows[g]

            def kblock(it, acc_t, kb=kb, vrow=vrow):
                k0 = it * _KUN
                a = list(acc_t)
                for kk in range(_KUN):
                    bc = _bcast_lane(vrow, k0 + kk)
                    for c in range(_NDC):
                        a[c] = a[c] + rbuf[kb + k0 + kk,
                                           pl.ds(c * _L, _L)] * bc
                return tuple(a)

            if nk >= _KUN:
                acc = lax.fori_loop(0, nk // _KUN, kblock, acc)
            for k in range(nk - nk % _KUN, nk):
                bc = _bcast_lane(vrow, k)
                acc = tuple(acc[c] + rbuf[kb + k, pl.ds(c * _L, _L)] * bc
                            for c in range(_NDC))
        for c in range(_NDC):
            out_v[qd, i, pl.ds(c * _L, _L)] = acc[c]

    def chunk_body(chunk, _):
        qd = lax.rem(chunk, 2)
        cbase = base + chunk * _CHUNK
        gbase = wid * (_BPW // _PAIR) + chunk * (_CHUNK // _PAIR)
        pltpu.sync_copy(idx_hbm.at[pl.ds(gbase, _CHUNK // _PAIR)], idx_v)
        pltpu.sync_copy(val_hbm.at[pl.ds(cbase, _CHUNK)], val_v)

        for qs in range(2):
            @pl.when(jnp.logical_and(chunk >= 2, qd == qs))
            def _(qs=qs):
                pltpu.make_async_copy(
                    out_v.at[qs],
                    out_hbm.at[pl.ds(cbase - 2 * _CHUNK, _CHUNK)],
                    osems[qs]).wait()

        for b in range(_NBUF):
            pltpu.async_copy(w_hbm.at[idx_v.at[b]], rows_v.at[b], sems[b])

        def ring_round(r, _):
            g0 = r * _NBUF
            for b in range(_NBUF):
                g = g0 + b
                pltpu.make_async_copy(
                    w_hbm.at[idx_v.at[g]], rows_v.at[b], sems[b]).wait()
                for p in range(_PAIR):
                    compute_row(g * _PAIR + p, p, qd, rows_v.at[b])
                gnext = g + _NBUF

                @pl.when(gnext < _CHUNK // _PAIR)
                def _():
                    pltpu.async_copy(
                        w_hbm.at[idx_v.at[gnext]], rows_v.at[b], sems[b])
            return 0

        lax.fori_loop(0, _CHUNK // _PAIR // _NBUF, ring_round, 0)
        for qs in range(2):
            @pl.when(qd == qs)
            def _(qs=qs):
                pltpu.async_copy(out_v.at[qs],
                                 out_hbm.at[pl.ds(cbase, _CHUNK)], osems[qs])
        return 0

    lax.fori_loop(0, _NCHUNK, chunk_body, 0)
    for chunk in (_NCHUNK - 2, _NCHUNK - 1):
        qs = chunk & 1
        pltpu.make_async_copy(
            out_v.at[qs],
            out_hbm.at[pl.ds(base + chunk * _CHUNK, _CHUNK)],
            osems[qs]).wait()


@functools.partial(
    pl.kernel,
    out_type=jax.ShapeDtypeStruct((_B, _D), jnp.float32),
    mesh=plsc.VectorSubcoreMesh(core_axis_name="c", subcore_axis_name="s"),
    scratch_types=[
        pltpu.VMEM((_CHUNK // _PAIR, _PAIR * _K), jnp.int32),
        pltpu.VMEM((_CHUNK, _KP), jnp.float32),
        pltpu.VMEM((_NBUF, _PAIR * _K, _D), jnp.float32),
        pltpu.VMEM((2, _CHUNK, _D), jnp.float32),
        pltpu.VMEM((_D,), jnp.float32),
    ] + [pltpu.SemaphoreType.DMA] * (_NBUF + 2),
)
def _ft_sc(idx_hbm, val_hbm, w_hbm, bias_hbm, out_hbm,
           idx_v, val_v, rows_v, out_v, bias_v, *sems):
    _tec_body(idx_hbm, val_hbm, w_hbm, bias_hbm, out_hbm,
              idx_v, val_v, rows_v, out_v, bias_v,
              sems[:_NBUF], sems[_NBUF:])


def kernel(feature_indices, feature_values, weight, bias):
    idx_grouped = feature_indices.reshape(_B // _PAIR, _PAIR * _K)
    vals_padded = jnp.pad(feature_values, ((0, 0), (0, _KP - _K)))
    return _ft_sc(idx_grouped, vals_padded, weight, bias)


# speedup vs baseline: 1.2431x; 1.0214x over previous
import functools

import jax
import jax.numpy as jnp
import numpy as np
from jax import lax
from jax.experimental import pallas as pl
from jax.experimental.pallas import tpu as pltpu
from jax.experimental.pallas import tpu_sc as plsc

_B = 16384
_K = 50
_D = 128
_L = 16
_NC = 2
_NS = 16
_NW = _NC * _NS
_BPW = _B // _NW
_CHUNK = 128
_NCHUNK = _BPW // _CHUNK
_NDC = _D // _L
_KP = 64
_NBUF = 4
_PAIR = 1
_KUN = 8

_BCAST_DNUMS = lax.GatherDimensionNumbers(
    offset_dims=(), collapsed_slice_dims=(0,), start_index_map=(0,))


def _bcast_lane(vec, lane):
    idx = jnp.full((_L,), lane, jnp.int32).reshape(_L, 1)
    return lax.gather(vec, idx, _BCAST_DNUMS, (1,),
                      mode=lax.GatherScatterMode.PROMISE_IN_BOUNDS)


def _tec_body(idx_hbm, val_hbm, w_hbm, bias_hbm, out_hbm,
              idx_v, val_v, rows_v, out_v, bias_v, sems, osems, ssems):
    cidx = lax.axis_index("c")
    sidx = lax.axis_index("s")
    wid = sidx * _NC + cidx
    base = wid * _BPW

    pltpu.sync_copy(bias_hbm, bias_v)
    bias_vecs = tuple(bias_v[pl.ds(c * _L, _L)] for c in range(_NDC))

    def compute_row(i, p, qd, rbuf):
        vrows = [val_v[qd, i, pl.ds(g * _L, _L)] for g in range(_KP // _L)]
        acc = tuple(bias_vecs)
        for g in range(_K // _L + 1):
            nk = _L if g < _K // _L else _K % _L
            kb = p * _K + g * _L
            vrow = vrows[g]

            def kblock(it, acc_t, kb=kb, vrow=vrow):
                k0 = it * _KUN
                a = list(acc_t)
                for kk in range(_KUN):
                    bc = _bcast_lane(vrow, k0 + kk)
                    for c in range(_NDC):
                        a[c] = a[c] + rbuf[kb + k0 + kk,
                                           pl.ds(c * _L, _L)] * bc
                return tuple(a)

            if nk >= _KUN:
                acc = lax.fori_loop(0, nk // _KUN, kblock, acc)
            for k in range(nk - nk % _KUN, nk):
                bc = _bcast_lane(vrow, k)
                acc = tuple(acc[c] + rbuf[kb + k, pl.ds(c * _L, _L)] * bc
                            for c in range(_NDC))
        for c in range(_NDC):
            out_v[qd, i, pl.ds(c * _L, _L)] = acc[c]

    gbase0 = wid * (_BPW // _PAIR)
    pltpu.async_copy(idx_hbm.at[pl.ds(gbase0, _CHUNK // _PAIR)],
                     idx_v.at[0], ssems[0])
    pltpu.async_copy(val_hbm.at[pl.ds(base, _CHUNK)], val_v.at[0], ssems[0])

    def chunk_body(chunk, _):
        qd = lax.rem(chunk, 2)
        cbase = base + chunk * _CHUNK
        gbase = wid * (_BPW // _PAIR) + chunk * (_CHUNK // _PAIR)

        for qs in range(2):
            @pl.when(qd == qs)
            def _(qs=qs):
                pltpu.make_async_copy(
                    idx_hbm.at[pl.ds(gbase, _CHUNK // _PAIR)],
                    idx_v.at[qs], ssems[qs]).wait()
                pltpu.make_async_copy(
                    val_hbm.at[pl.ds(cbase, _CHUNK)],
                    val_v.at[qs], ssems[qs]).wait()

                @pl.when(chunk + 1 < _NCHUNK)
                def _():
                    pltpu.async_copy(
                        idx_hbm.at[pl.ds(gbase + _CHUNK // _PAIR,
                                         _CHUNK // _PAIR)],
                        idx_v.at[1 - qs], ssems[1 - qs])
                    pltpu.async_copy(
                        val_hbm.at[pl.ds(cbase + _CHUNK, _CHUNK)],
                        val_v.at[1 - qs], ssems[1 - qs])

        for qs in range(2):
            @pl.when(jnp.logical_and(chunk >= 2, qd == qs))
            def _(qs=qs):
                pltpu.make_async_copy(
                    out_v.at[qs],
                    out_hbm.at[pl.ds(cbase - 2 * _CHUNK, _CHUNK)],
                    osems[qs]).wait()

        for b in range(_NBUF):
            pltpu.async_copy(w_hbm.at[idx_v.at[qd, b]], rows_v.at[b],
                             sems[b])

        def ring_round(r, _):
            g0 = r * _NBUF
            for b in range(_NBUF):
                g = g0 + b
                pltpu.make_async_copy(
                    w_hbm.at[idx_v.at[qd, g]], rows_v.at[b], sems[b]).wait()
                for p in range(_PAIR):
                    compute_row(g * _PAIR + p, p, qd, rows_v.at[b])
                gnext = g + _NBUF

                @pl.when(gnext < _CHUNK // _PAIR)
                def _():
                    pltpu.async_copy(
                        w_hbm.at[idx_v.at[qd, gnext]], rows_v.at[b], sems[b])
            return 0

        lax.fori_loop(0, _CHUNK // _PAIR // _NBUF, ring_round, 0)
        for qs in range(2):
            @pl.when(qd == qs)
            def _(qs=qs):
                pltpu.async_copy(out_v.at[qs],
                                 out_hbm.at[pl.ds(cbase, _CHUNK)], osems[qs])
        return 0

    lax.fori_loop(0, _NCHUNK, chunk_body, 0)
    for chunk in (_NCHUNK - 2, _NCHUNK - 1):
        qs = chunk & 1
        pltpu.make_async_copy(
            out_v.at[qs],
            out_hbm.at[pl.ds(base + chunk * _CHUNK, _CHUNK)],
            osems[qs]).wait()


@functools.partial(
    pl.kernel,
    out_type=jax.ShapeDtypeStruct((_B, _D), jnp.float32),
    mesh=plsc.VectorSubcoreMesh(core_axis_name="c", subcore_axis_name="s"),
    scratch_types=[
        pltpu.VMEM((2, _CHUNK // _PAIR, _PAIR * _K), jnp.int32),
        pltpu.VMEM((2, _CHUNK, _KP), jnp.float32),
        pltpu.VMEM((_NBUF, _PAIR * _K, _D), jnp.float32),
        pltpu.VMEM((2, _CHUNK, _D), jnp.float32),
        pltpu.VMEM((_D,), jnp.float32),
    ] + [pltpu.SemaphoreType.DMA] * (_NBUF + 4),
)
def _ft_sc(idx_hbm, val_hbm, w_hbm, bias_hbm, out_hbm,
           idx_v, val_v, rows_v, out_v, bias_v, *sems):
    _tec_body(idx_hbm, val_hbm, w_hbm, bias_hbm, out_hbm,
              idx_v, val_v, rows_v, out_v, bias_v,
              sems[:_NBUF], sems[_NBUF:_NBUF + 2], sems[_NBUF + 2:])


def kernel(feature_indices, feature_values, weight, bias):
    idx_grouped = feature_indices.reshape(_B // _PAIR, _PAIR * _K)
    vals_padded = jnp.pad(feature_values, ((0, 0), (0, _KP - _K)))
    return _ft_sc(idx_grouped, vals_padded, weight, bias)
